# Initial kernel scaffold; baseline (speedup 1.0000x reference)
import jax
import jax.numpy as jnp
from jax.experimental import pallas as pl


def kernel(node0, node1, node2):
    raise NotImplementedError("write your pallas kernel here")


# speedup vs baseline: 3.1399x; 3.1399x over previous
import jax
import jax.numpy as jnp
from jax.experimental import pallas as pl
from jax.experimental.pallas import tpu as pltpu

_N, _D = 100000, 128
_BLOCK = 2000


def _node_kernel(n0_ref, n1_ref, n2_ref, feat_ref, sums_ref):
    i = pl.program_id(0)
    a = n0_ref[...]
    b = n1_ref[...]
    c = n2_ref[...]
    p01 = a * b
    p12 = b * c
    feat_ref[0, :, 0:_D] = a
    feat_ref[0, :, _D:2 * _D] = b
    feat_ref[0, :, 2 * _D:3 * _D] = p01
    feat_ref[1, :, 0:_D] = b
    feat_ref[1, :, _D:2 * _D] = c
    feat_ref[1, :, 2 * _D:3 * _D] = p12

    @pl.when(i == 0)
    def _():
        sums_ref[0] = 0.0
        sums_ref[1] = 0.0

    sums_ref[0] += jnp.sum(p01)
    sums_ref[1] += jnp.sum(p12)


def kernel(node0, node1, node2):
    n = node0.shape[0]
    grid = n // _BLOCK
    feats, sums = pl.pallas_call(
        _node_kernel,
        grid=(grid,),
        in_specs=[
            pl.BlockSpec((_BLOCK, _D), lambda i: (i, 0)),
            pl.BlockSpec((_BLOCK, _D), lambda i: (i, 0)),
            pl.BlockSpec((_BLOCK, _D), lambda i: (i, 0)),
        ],
        out_specs=[
            pl.BlockSpec((2, _BLOCK, 3 * _D), lambda i: (0, i, 0)),
            pl.BlockSpec(memory_space=pltpu.SMEM),
        ],
        out_shape=[
            jax.ShapeDtypeStruct((2, n, 3 * _D), jnp.float32),
            jax.ShapeDtypeStruct((2,), jnp.float32),
        ],
    )(node0, node1, node2)
    return feats, sums


# speedup vs baseline: 3.2432x; 1.0329x over previous
import jax
import jax.numpy as jnp
from jax.experimental import pallas as pl
from jax.experimental.pallas import tpu as pltpu

_N, _D = 100000, 128
_BLOCK = 4000


def _node_kernel(n0_ref, n1_ref, n2_ref, feat_ref, sums_ref):
    i = pl.program_id(0)
    a = n0_ref[...]
    b = n1_ref[...]
    c = n2_ref[...]
    p01 = a * b
    p12 = b * c
    feat_ref[0, :, 0:_D] = a
    feat_ref[0, :, _D:2 * _D] = b
    feat_ref[0, :, 2 * _D:3 * _D] = p01
    feat_ref[1, :, 0:_D] = b
    feat_ref[1, :, _D:2 * _D] = c
    feat_ref[1, :, 2 * _D:3 * _D] = p12

    @pl.when(i == 0)
    def _():
        sums_ref[0] = 0.0
        sums_ref[1] = 0.0

    sums_ref[0] += jnp.sum(p01)
    sums_ref[1] += jnp.sum(p12)


def kernel(node0, node1, node2):
    n = node0.shape[0]
    grid = n // _BLOCK
    feats, sums = pl.pallas_call(
        _node_kernel,
        grid=(grid,),
        in_specs=[
            pl.BlockSpec((_BLOCK, _D), lambda i: (i, 0)),
            pl.BlockSpec((_BLOCK, _D), lambda i: (i, 0)),
            pl.BlockSpec((_BLOCK, _D), lambda i: (i, 0)),
        ],
        out_specs=[
            pl.BlockSpec((2, _BLOCK, 3 * _D), lambda i: (0, i, 0)),
            pl.BlockSpec(memory_space=pltpu.SMEM),
        ],
        out_shape=[
            jax.ShapeDtypeStruct((2, n, 3 * _D), jnp.float32),
            jax.ShapeDtypeStruct((2,), jnp.float32),
        ],
    )(node0, node1, node2)
    return feats, sums


# speedup vs baseline: 3.2629x; 1.0061x over previous
import jax
import jax.numpy as jnp
from jax.experimental import pallas as pl
from jax.experimental.pallas import tpu as pltpu

_N, _D = 100000, 128
_BLOCK = 5000


def _node_kernel(n0_ref, n1_ref, n2_ref, feat_ref, sums_ref):
    i = pl.program_id(0)
    a = n0_ref[...]
    b = n1_ref[...]
    c = n2_ref[...]
    p01 = a * b
    p12 = b * c
    feat_ref[0, :, 0:_D] = a
    feat_ref[0, :, _D:2 * _D] = b
    feat_ref[0, :, 2 * _D:3 * _D] = p01
    feat_ref[1, :, 0:_D] = b
    feat_ref[1, :, _D:2 * _D] = c
    feat_ref[1, :, 2 * _D:3 * _D] = p12

    @pl.when(i == 0)
    def _():
        sums_ref[0] = 0.0
        sums_ref[1] = 0.0

    sums_ref[0] += jnp.sum(p01)
    sums_ref[1] += jnp.sum(p12)


def kernel(node0, node1, node2):
    n = node0.shape[0]
    grid = n // _BLOCK
    feats, sums = pl.pallas_call(
        _node_kernel,
        grid=(grid,),
        in_specs=[
            pl.BlockSpec((_BLOCK, _D), lambda i: (i, 0)),
            pl.BlockSpec((_BLOCK, _D), lambda i: (i, 0)),
            pl.BlockSpec((_BLOCK, _D), lambda i: (i, 0)),
        ],
        out_specs=[
            pl.BlockSpec((2, _BLOCK, 3 * _D), lambda i: (0, i, 0)),
            pl.BlockSpec(memory_space=pltpu.SMEM),
        ],
        out_shape=[
            jax.ShapeDtypeStruct((2, n, 3 * _D), jnp.float32),
            jax.ShapeDtypeStruct((2,), jnp.float32),
        ],
    )(node0, node1, node2)
    return feats, sums
